# Initial kernel scaffold; baseline (speedup 1.0000x reference)
import jax
import jax.numpy as jnp
from jax.experimental import pallas as pl


def kernel(input):
    raise NotImplementedError("write your pallas kernel here")


# speedup vs baseline: 181.9053x; 181.9053x over previous
import functools

import jax
import jax.numpy as jnp
from jax import lax
from jax.experimental import pallas as pl
from jax.experimental.pallas import tpu as pltpu
from jax.experimental.pallas import tpu_sc as plsc

B = 128
N = 512
L = 16
NC = 2
NS = 16
NW = NC * NS
RPW = B // NW
NCHUNK = N // L

_LN2 = 0.6931471805599453


def _vlog(k_f32):
    i = plsc.bitcast(k_f32, jnp.int32)
    e = (i >> 23) - 127
    m = plsc.bitcast((i & 0x007FFFFF) | 0x3F800000, jnp.float32)
    t = (m - 1.0) / (m + 1.0)
    t2 = t * t
    p = 1.0 / 11.0
    for c in (1.0 / 9.0, 1.0 / 7.0, 1.0 / 5.0, 1.0 / 3.0, 1.0):
        p = p * t2 + c
    return e.astype(jnp.float32) * _LN2 + 2.0 * t * p


def _vsqrt(v):
    i = plsc.bitcast(jnp.maximum(v, 1e-38), jnp.int32)
    y = plsc.bitcast(0x5F3759DF - (i >> 1), jnp.float32)
    for _ in range(3):
        y = y * (1.5 - 0.5 * v * y * y)
    s = v * y
    return jnp.where(v > 0.0, s, 0.0)


def _tour_body(xs_hbm, ys_hbm, cost_hbm, ll_hbm, xv, yv, cv, lv):
    wid = lax.axis_index("c") * NS + lax.axis_index("s")
    pltpu.sync_copy(xs_hbm.at[wid], xv.at[pl.ds(0, RPW * N)])
    pltpu.sync_copy(ys_hbm.at[wid], yv.at[pl.ds(0, RPW * N)])
    lane = lax.iota(jnp.int32, L)
    zero = jnp.zeros((L,), jnp.float32)
    xv[pl.ds(RPW * N, L)] = zero
    yv[pl.ds(RPW * N, L)] = zero

    cvec = zero
    for r in range(RPW):
        base = r * N

        def edge_chunk(c, acc, base=base):
            off = base + c * L
            dx = xv[pl.ds(off, L)] - xv[pl.ds(off + 1, L)]
            dy = yv[pl.ds(off, L)] - yv[pl.ds(off + 1, L)]
            return acc + _vsqrt(dx * dx + dy * dy)

        acc = lax.fori_loop(0, NCHUNK - 1, edge_chunk, zero)
        off = base + N - L
        ax = xv[pl.ds(off, L)]
        ay = yv[pl.ds(off, L)]
        firstx = xv[pl.ds(base, L)][0]
        firsty = yv[pl.ds(base, L)][0]
        bx = jnp.where(lane == L - 1, firstx, xv[pl.ds(off + 1, L)])
        by = jnp.where(lane == L - 1, firsty, yv[pl.ds(off + 1, L)])
        dx = ax - bx
        dy = ay - by
        acc = acc + _vsqrt(dx * dx + dy * dy)
        cvec = jnp.where(lane == r, jnp.sum(acc), cvec)
    cv[...] = cvec

    def ll_chunk(c, acc):
        k = (lane + (c * L + 1)).astype(jnp.float32)
        return acc + _vlog(k)

    ll_acc = lax.fori_loop(0, NCHUNK, ll_chunk, zero)
    lv[...] = jnp.full((L,), -jnp.sum(ll_acc), jnp.float32)

    pltpu.sync_copy(cv, cost_hbm.at[wid])
    pltpu.sync_copy(lv, ll_hbm.at[wid])


@functools.partial(
    pl.kernel,
    out_type=(
        jax.ShapeDtypeStruct((NW, L), jnp.float32),
        jax.ShapeDtypeStruct((NW, L), jnp.float32),
    ),
    mesh=plsc.VectorSubcoreMesh(
        core_axis_name="c", subcore_axis_name="s", num_cores=NC, num_subcores=NS
    ),
    scratch_types=(
        pltpu.VMEM((RPW * N + L,), jnp.float32),
        pltpu.VMEM((RPW * N + L,), jnp.float32),
        pltpu.VMEM((L,), jnp.float32),
        pltpu.VMEM((L,), jnp.float32),
    ),
    compiler_params=pltpu.CompilerParams(needs_layout_passes=False),
)
def _tour_kernel(xs_hbm, ys_hbm, cost_hbm, ll_hbm, xv, yv, cv, lv):
    _tour_body(xs_hbm, ys_hbm, cost_hbm, ll_hbm, xv, yv, cv, lv)


def kernel(input):
    xs = input[:, :, 0].reshape(NW, RPW * N)
    ys = input[:, :, 1].reshape(NW, RPW * N)
    cost_t, ll_t = _tour_kernel(xs, ys)
    return cost_t[:, :RPW].reshape(B), ll_t[:, :RPW].reshape(B)
